# baseline (speedup 1.0000x reference)
import jax
import jax.numpy as jnp
from jax import lax
from jax.experimental import pallas as pl
from jax.experimental.pallas import tpu as pltpu
from jax.experimental.pallas import tpu_sc as plsc

NC, NS = 2, 16
NW = NC * NS
K1 = 4
K2 = 4

_MESH = dict(core_axis_name="c", subcore_axis_name="s")


def _worker_id():
    return lax.axis_index("s") * NC + lax.axis_index("c")


def _phase1(tt):
    D, V = tt.shape
    nblk = V // 128
    tail = V % 128
    blk_w = nblk // NW
    extra = nblk % NW

    @pl.kernel(
        out_type=jax.ShapeDtypeStruct((V // 2, 128), jnp.float32),
        mesh=plsc.VectorSubcoreMesh(**_MESH),
        scratch_types=[
            pltpu.VMEM((K1, 64, 128), jnp.float32),
            pltpu.VMEM((K1, 64, 128), jnp.float32),
            pltpu.SemaphoreType.DMA((K1,)),
            pltpu.SemaphoreType.DMA((K1,)),
        ],
        compiler_params=pltpu.CompilerParams(use_tc_tiling_on_sc=True, needs_layout_passes=False),
    )
    def p1(tt_hbm, t2_hbm, in_ring, out_ring, rsem, wsem):
        wid = _worker_id()
        base = wid * blk_w
        iota = lax.iota(jnp.int32, 16)
        rows4 = [iota + 16 * k for k in range(4)]

        def read_start(b, blk):
            pltpu.make_async_copy(
                tt_hbm.at[:, pl.ds(pl.multiple_of(blk * 128, 128), 128)], in_ring.at[b], rsem.at[b]
            ).start()

        def read_wait(b, blk):
            pltpu.make_async_copy(
                tt_hbm.at[:, pl.ds(pl.multiple_of(blk * 128, 128), 128)], in_ring.at[b], rsem.at[b]
            ).wait()

        def write_start(b, blk):
            pltpu.make_async_copy(
                out_ring.at[b], t2_hbm.at[pl.ds(blk * 64, 64)], wsem.at[b]
            ).start()

        def write_wait(b, blk):
            pltpu.make_async_copy(
                out_ring.at[b], t2_hbm.at[pl.ds(blk * 64, 64)], wsem.at[b]
            ).wait()

        def transpose_block(b, n_rows, vbase):
            @pl.loop(0, n_rows, step=8)
            def _(r0):
                for dr in range(8):
                    r = r0 + dr
                    for j in range(8):
                        col = jnp.full((16,), vbase + 2 * r + (1 if j >= 4 else 0),
                                       dtype=jnp.int32)
                        vals = plsc.load_gather(in_ring.at[b], [rows4[j % 4], col])
                        out_ring[b, r, pl.ds(16 * j, 16)] = vals

        for b in range(K1):
            read_start(b, base + b)

        @pl.loop(0, blk_w - K1, step=K1)
        def _(j0):
            for b in range(K1):
                blk = base + j0 + b
                read_wait(b, blk)

                @pl.when(j0 + b >= K1)
                def _():
                    write_wait(b, blk)

                transpose_block(b, 64, 0)
                write_start(b, blk)
                read_start(b, blk + K1)

        for b in range(K1):
            blk = base + blk_w - K1 + b
            read_wait(b, blk)
            write_wait(b, blk)
            transpose_block(b, 64, 0)
            write_start(b, blk)
            write_wait(b, blk)

        @pl.when(wid < extra)
        def _():
            blk = NW * blk_w + wid
            read_start(0, blk)
            read_wait(0, blk)
            transpose_block(0, 64, 0)
            write_start(0, blk)
            write_wait(0, blk)

        if tail:
            @pl.when(wid == extra)
            def _():
                last = pltpu.make_async_copy(
                    tt_hbm.at[:, pl.ds(pl.multiple_of(V - tail, 128), 128)],
                    in_ring.at[1], rsem.at[1]
                )
                last.start()
                last.wait()
                iota_l = lax.iota(jnp.int32, 16)
                rows4_l = [iota_l + 16 * k for k in range(4)]

                @pl.loop(0, tail // 2, step=8)
                def _(r0):
                    for dr in range(8):
                        r = r0 + dr
                        for j in range(8):
                            col = jnp.full(
                                (16,), 2 * r + (1 if j >= 4 else 0),
                                dtype=jnp.int32)
                            vals = plsc.load_gather(
                                in_ring.at[1], [rows4_l[j % 4], col])
                            out_ring[1, r, pl.ds(16 * j, 16)] = vals

                wlast = pltpu.make_async_copy(
                    out_ring.at[1, pl.ds(0, tail // 2)],
                    t2_hbm.at[pl.ds((V - tail) // 2, tail // 2)], wsem.at[1])
                wlast.start()
                wlast.wait()

    return p1(tt)


def _phase2(t2r, idx_lin, H, B):
    V, D = t2r.shape
    nunit = idx_lin.shape[0]
    upw = nunit // NW
    bph = B // 128

    @pl.kernel(
        out_type=jax.ShapeDtypeStruct((H, D, B), jnp.float32),
        mesh=plsc.VectorSubcoreMesh(**_MESH),
        scratch_types=[
            pltpu.VMEM((upw, 128), jnp.int32),
            pltpu.VMEM((K2, 128, 64), jnp.float32),
            pltpu.VMEM((K2, 64, 128), jnp.float32),
            pltpu.SemaphoreType.DMA((K2,)),
            pltpu.SemaphoreType.DMA((K2,)),
            pltpu.SemaphoreType.DMA,
        ],
        compiler_params=pltpu.CompilerParams(use_tc_tiling_on_sc=False, needs_layout_passes=False),
    )
    def p2(t2_hbm, idx_hbm, p_hbm, idx_v, g_ring, t_ring, gsem, wsem, isem):
        wid = _worker_id()
        ubase = wid * upw
        pltpu.async_copy(idx_hbm.at[pl.ds(ubase, upw)], idx_v, isem).wait()
        iota = lax.iota(jnp.int32, 16)
        rows8 = [iota + 16 * k for k in range(8)]

        def gather_start(b, l):
            pltpu.make_async_copy(
                t2_hbm.at[idx_v.at[l]], g_ring.at[b], gsem.at[b]
            ).start()

        def gather_wait(b, l):
            pltpu.make_async_copy(
                t2_hbm.at[idx_v.at[l]], g_ring.at[b], gsem.at[b]
            ).wait()

        def dst(l):
            u = ubase + l
            h = u // bph
            b0 = (u % bph) * 128
            return p_hbm.at[h, :, pl.ds(b0, 128)]

        def write_start(b, l):
            pltpu.make_async_copy(t_ring.at[b], dst(l), wsem.at[b]).start()

        def write_wait(b, l):
            pltpu.make_async_copy(t_ring.at[b], dst(l), wsem.at[b]).wait()

        def transpose_unit(b):
            @pl.loop(0, 64, step=8)
            def _(d0):
                for dd in range(8):
                    d = d0 + dd
                    col = jnp.full((16,), d, dtype=jnp.int32)
                    for j in range(8):
                        vals = plsc.load_gather(g_ring.at[b], [rows8[j], col])
                        t_ring[b, d, pl.ds(16 * j, 16)] = vals

        for b in range(K2):
            gather_start(b, b)

        @pl.loop(0, upw - K2, step=K2)
        def _(l0):
            for b in range(K2):
                l = l0 + b
                gather_wait(b, l)

                @pl.when(l0 + b >= K2)
                def _():
                    write_wait(b, l)

                transpose_unit(b)
                write_start(b, l)
                gather_start(b, l + K2)

        for b in range(K2):
            l = upw - K2 + b
            gather_wait(b, l)
            write_wait(b, l)
            transpose_unit(b)
            write_start(b, l)
            write_wait(b, l)

    return p2(t2r, idx_lin)


def kernel(indices, table):
    B, H = indices.shape
    V, D = table.shape
    tt = table.T
    t2 = _phase1(tt)
    t2r = t2.reshape(V, D)
    idx_lin = indices.T.reshape(B * H // 128, 128).astype(jnp.int32)
    p = _phase2(t2r, idx_lin, H, B)
    return jnp.transpose(p, (2, 0, 1))


# speedup vs baseline: 1.2316x; 1.2316x over previous
import jax
import jax.numpy as jnp
from jax import lax
from jax.experimental import pallas as pl
from jax.experimental.pallas import tpu as pltpu
from jax.experimental.pallas import tpu_sc as plsc

NC, NS = 2, 16
NW = NC * NS
K1 = 4
K2 = 4

_MESH = dict(core_axis_name="c", subcore_axis_name="s")


def _worker_id():
    return lax.axis_index("s") * NC + lax.axis_index("c")


def _phase1(tt):
    D, V = tt.shape
    nblk = V // 128
    tail = V % 128
    blk_w = nblk // NW
    extra = nblk % NW

    @pl.kernel(
        out_type=jax.ShapeDtypeStruct((V // 2, 128), jnp.float32),
        mesh=plsc.VectorSubcoreMesh(**_MESH),
        scratch_types=[
            pltpu.VMEM((K1, 64, 129), jnp.float32),
            pltpu.VMEM((K1, 64, 128), jnp.float32),
            pltpu.SemaphoreType.DMA((K1,)),
            pltpu.SemaphoreType.DMA((K1,)),
        ],
        compiler_params=pltpu.CompilerParams(use_tc_tiling_on_sc=True, needs_layout_passes=False),
    )
    def p1(tt_hbm, t2_hbm, in_ring, out_ring, rsem, wsem):
        wid = _worker_id()
        base = wid * blk_w
        iota = lax.iota(jnp.int32, 16)
        rows4 = [iota + 16 * k for k in range(4)]

        def read_start(b, blk):
            pltpu.make_async_copy(
                tt_hbm.at[:, pl.ds(pl.multiple_of(blk * 128, 128), 128)],
                in_ring.at[b, :, pl.ds(0, 128)], rsem.at[b]
            ).start()

        def read_wait(b, blk):
            pltpu.make_async_copy(
                tt_hbm.at[:, pl.ds(pl.multiple_of(blk * 128, 128), 128)],
                in_ring.at[b, :, pl.ds(0, 128)], rsem.at[b]
            ).wait()

        def write_start(b, blk):
            pltpu.make_async_copy(
                out_ring.at[b], t2_hbm.at[pl.ds(blk * 64, 64)], wsem.at[b]
            ).start()

        def write_wait(b, blk):
            pltpu.make_async_copy(
                out_ring.at[b], t2_hbm.at[pl.ds(blk * 64, 64)], wsem.at[b]
            ).wait()

        def transpose_block(b, n_rows, vbase):
            @pl.loop(0, n_rows, step=8)
            def _(r0):
                for dr in range(8):
                    r = r0 + dr
                    for j in range(8):
                        col = jnp.full((16,), vbase + 2 * r + (1 if j >= 4 else 0),
                                       dtype=jnp.int32)
                        vals = plsc.load_gather(in_ring.at[b], [rows4[j % 4], col])
                        out_ring[b, r, pl.ds(16 * j, 16)] = vals

        for b in range(K1):
            read_start(b, base + b)

        @pl.loop(0, blk_w - K1, step=K1)
        def _(j0):
            for b in range(K1):
                blk = base + j0 + b
                read_wait(b, blk)

                @pl.when(j0 + b >= K1)
                def _():
                    write_wait(b, blk)

                transpose_block(b, 64, 0)
                write_start(b, blk)
                read_start(b, blk + K1)

        for b in range(K1):
            blk = base + blk_w - K1 + b
            read_wait(b, blk)
            write_wait(b, blk)
            transpose_block(b, 64, 0)
            write_start(b, blk)
            write_wait(b, blk)

        @pl.when(wid < extra)
        def _():
            blk = NW * blk_w + wid
            read_start(0, blk)
            read_wait(0, blk)
            transpose_block(0, 64, 0)
            write_start(0, blk)
            write_wait(0, blk)

        if tail:
            @pl.when(wid == extra)
            def _():
                last = pltpu.make_async_copy(
                    tt_hbm.at[:, pl.ds(pl.multiple_of(V - tail, 128), 128)],
                    in_ring.at[1, :, pl.ds(0, 128)], rsem.at[1]
                )
                last.start()
                last.wait()
                iota_l = lax.iota(jnp.int32, 16)
                rows4_l = [iota_l + 16 * k for k in range(4)]

                @pl.loop(0, tail // 2, step=8)
                def _(r0):
                    for dr in range(8):
                        r = r0 + dr
                        for j in range(8):
                            col = jnp.full(
                                (16,), 2 * r + (1 if j >= 4 else 0),
                                dtype=jnp.int32)
                            vals = plsc.load_gather(
                                in_ring.at[1], [rows4_l[j % 4], col])
                            out_ring[1, r, pl.ds(16 * j, 16)] = vals

                wlast = pltpu.make_async_copy(
                    out_ring.at[1, pl.ds(0, tail // 2)],
                    t2_hbm.at[pl.ds((V - tail) // 2, tail // 2)], wsem.at[1])
                wlast.start()
                wlast.wait()

    return p1(tt)


def _phase2(t2r, idx_lin, H, B):
    V, D = t2r.shape
    nunit = idx_lin.shape[0]
    upw = nunit // NW
    bph = B // 128

    @pl.kernel(
        out_type=jax.ShapeDtypeStruct((H, D, B), jnp.float32),
        mesh=plsc.VectorSubcoreMesh(**_MESH),
        scratch_types=[
            pltpu.VMEM((upw, 128), jnp.int32),
            pltpu.VMEM((K2, 128, 64), jnp.float32),
            pltpu.VMEM((64, 129), jnp.float32),
            pltpu.VMEM((K2, 64, 128), jnp.float32),
            pltpu.SemaphoreType.DMA((K2,)),
            pltpu.SemaphoreType.DMA((K2,)),
            pltpu.SemaphoreType.DMA,
        ],
        compiler_params=pltpu.CompilerParams(use_tc_tiling_on_sc=False, needs_layout_passes=False),
    )
    def p2(t2_hbm, idx_hbm, p_hbm, idx_v, g_ring, t_pad, t_ring, gsem, wsem, isem):
        wid = _worker_id()
        ubase = wid * upw
        pltpu.async_copy(idx_hbm.at[pl.ds(ubase, upw)], idx_v, isem).wait()
        iota = lax.iota(jnp.int32, 16)
        rows8 = [iota + 16 * k for k in range(8)]

        def gather_start(b, l):
            pltpu.make_async_copy(
                t2_hbm.at[idx_v.at[l]], g_ring.at[b], gsem.at[b]
            ).start()

        def gather_wait(b, l):
            pltpu.make_async_copy(
                t2_hbm.at[idx_v.at[l]], g_ring.at[b], gsem.at[b]
            ).wait()

        def dst(l):
            u = ubase + l
            h = u // bph
            b0 = (u % bph) * 128
            return p_hbm.at[h, :, pl.ds(b0, 128)]

        def write_start(b, l):
            pltpu.make_async_copy(t_ring.at[b], dst(l), wsem.at[b]).start()

        def write_wait(b, l):
            pltpu.make_async_copy(t_ring.at[b], dst(l), wsem.at[b]).wait()

        def transpose_unit(b):
            @pl.loop(0, 128, step=8)
            def _(i0):
                for di in range(8):
                    i = i0 + di
                    col = jnp.full((16,), i, dtype=jnp.int32)
                    for j in range(4):
                        vals = g_ring[b, i, pl.ds(16 * j, 16)]
                        plsc.store_scatter(t_pad, [rows8[j], col], vals)

            @pl.loop(0, 64, step=8)
            def _(d0):
                for dd in range(8):
                    d = d0 + dd
                    for j in range(8):
                        t_ring[b, d, pl.ds(16 * j, 16)] = t_pad[d, pl.ds(16 * j, 16)]

        for b in range(K2):
            gather_start(b, b)

        @pl.loop(0, upw - K2, step=K2)
        def _(l0):
            for b in range(K2):
                l = l0 + b
                gather_wait(b, l)

                @pl.when(l0 + b >= K2)
                def _():
                    write_wait(b, l)

                transpose_unit(b)
                write_start(b, l)
                gather_start(b, l + K2)

        for b in range(K2):
            l = upw - K2 + b
            gather_wait(b, l)
            write_wait(b, l)
            transpose_unit(b)
            write_start(b, l)
            write_wait(b, l)

    return p2(t2r, idx_lin)


def kernel(indices, table):
    B, H = indices.shape
    V, D = table.shape
    tt = table.T
    t2 = _phase1(tt)
    t2r = t2.reshape(V, D)
    idx_lin = indices.T.reshape(B * H // 128, 128).astype(jnp.int32)
    p = _phase2(t2r, idx_lin, H, B)
    return jnp.transpose(p, (2, 0, 1))


# speedup vs baseline: 5.3324x; 4.3297x over previous
import jax
import jax.numpy as jnp
from jax import lax
from jax.experimental import pallas as pl
from jax.experimental.pallas import tpu as pltpu
from jax.experimental.pallas import tpu_sc as plsc

NC, NS = 2, 16
NW = NC * NS
K1 = 4
K2 = 4
PAD = 129

_MESH = dict(core_axis_name="c", subcore_axis_name="s")


def _worker_id():
    return lax.axis_index("s") * NC + lax.axis_index("c")


def _phase1(tt):
    D, V = tt.shape
    nblk = V // 128
    tail = V % 128
    blk_w = nblk // NW
    extra = nblk % NW

    @pl.kernel(
        out_type=jax.ShapeDtypeStruct((V // 2, 128), jnp.float32),
        mesh=plsc.VectorSubcoreMesh(**_MESH),
        scratch_types=[
            pltpu.VMEM((K1, 64, 128), jnp.float32),
            pltpu.VMEM((64 * PAD,), jnp.float32),
            pltpu.VMEM((K1, 64, 128), jnp.float32),
            pltpu.SemaphoreType.DMA((K1,)),
            pltpu.SemaphoreType.DMA((K1,)),
        ],
        compiler_params=pltpu.CompilerParams(
            use_tc_tiling_on_sc=True, needs_layout_passes=False),
    )
    def p1(tt_hbm, t2_hbm, in_ring, o_pad, out_ring, rsem, wsem):
        wid = _worker_id()
        base = wid * blk_w
        iota = lax.iota(jnp.int32, 16)
        base_j = [(8 * j + iota // 2) * PAD + (iota % 2) * 64 for j in range(8)]

        def read_start(b, blk):
            pltpu.make_async_copy(
                tt_hbm.at[:, pl.ds(pl.multiple_of(blk * 128, 128), 128)],
                in_ring.at[b], rsem.at[b]
            ).start()

        def read_wait(b, blk):
            pltpu.make_async_copy(
                tt_hbm.at[:, pl.ds(pl.multiple_of(blk * 128, 128), 128)],
                in_ring.at[b], rsem.at[b]
            ).wait()

        def write_start(b, blk):
            pltpu.make_async_copy(
                out_ring.at[b], t2_hbm.at[pl.ds(blk * 64, 64)], wsem.at[b]
            ).start()

        def write_wait(b, blk):
            pltpu.make_async_copy(
                out_ring.at[b], t2_hbm.at[pl.ds(blk * 64, 64)], wsem.at[b]
            ).wait()

        def transpose_block(b, n_rows):
            nj = n_rows // 8

            @plsc.parallel_loop(0, 64, unroll=4)
            def _(d):
                for j in range(nj):
                    vals = in_ring[b, d, pl.ds(16 * j, 16)]
                    plsc.store_scatter(o_pad, [base_j[j] + d], vals)

            @plsc.parallel_loop(0, n_rows, unroll=4)
            def _(r):
                for j in range(8):
                    out_ring[b, r, pl.ds(16 * j, 16)] = o_pad[
                        pl.ds(r * PAD + 16 * j, 16)]

        for b in range(K1):
            read_start(b, base + b)

        @pl.loop(0, blk_w - K1, step=K1)
        def _(j0):
            for b in range(K1):
                blk = base + j0 + b
                read_wait(b, blk)

                @pl.when(j0 + b >= K1)
                def _():
                    write_wait(b, blk)

                transpose_block(b, 64)
                write_start(b, blk)
                read_start(b, blk + K1)

        for b in range(K1):
            blk = base + blk_w - K1 + b
            read_wait(b, blk)
            write_wait(b, blk)
            transpose_block(b, 64)
            write_start(b, blk)
            write_wait(b, blk)

        @pl.when(wid < extra)
        def _():
            blk = NW * blk_w + wid
            read_start(0, blk)
            read_wait(0, blk)
            transpose_block(0, 64)
            write_start(0, blk)
            write_wait(0, blk)

        if tail:
            @pl.when(wid == extra)
            def _():
                last = pltpu.make_async_copy(
                    tt_hbm.at[:, pl.ds(pl.multiple_of(V - tail, 128), 128)],
                    in_ring.at[1], rsem.at[1]
                )
                last.start()
                last.wait()
                transpose_block(1, tail // 2)
                wlast = pltpu.make_async_copy(
                    out_ring.at[1, pl.ds(0, tail // 2)],
                    t2_hbm.at[pl.ds((V - tail) // 2, tail // 2)], wsem.at[1])
                wlast.start()
                wlast.wait()

    return p1(tt)


def _phase2(t2r, idx_lin, H, B):
    V, D = t2r.shape
    nunit = idx_lin.shape[0]
    upw = nunit // NW
    bph = B // 128

    @pl.kernel(
        out_type=jax.ShapeDtypeStruct((H, D, B), jnp.float32),
        mesh=plsc.VectorSubcoreMesh(**_MESH),
        scratch_types=[
            pltpu.VMEM((upw, 128), jnp.int32),
            pltpu.VMEM((K2, 128, 64), jnp.float32),
            pltpu.VMEM((64, PAD), jnp.float32),
            pltpu.VMEM((K2, 64, 128), jnp.float32),
            pltpu.SemaphoreType.DMA((K2,)),
            pltpu.SemaphoreType.DMA((K2,)),
            pltpu.SemaphoreType.DMA,
        ],
        compiler_params=pltpu.CompilerParams(
            use_tc_tiling_on_sc=False, needs_layout_passes=False),
    )
    def p2(t2_hbm, idx_hbm, p_hbm, idx_v, g_ring, t_pad, t_ring, gsem, wsem, isem):
        wid = _worker_id()
        ubase = wid * upw
        pltpu.async_copy(idx_hbm.at[pl.ds(ubase, upw)], idx_v, isem).wait()
        iota = lax.iota(jnp.int32, 16)
        rows8 = [iota + 16 * k for k in range(8)]

        def gather_start(b, l):
            pltpu.make_async_copy(
                t2_hbm.at[idx_v.at[l]], g_ring.at[b], gsem.at[b]
            ).start()

        def gather_wait(b, l):
            pltpu.make_async_copy(
                t2_hbm.at[idx_v.at[l]], g_ring.at[b], gsem.at[b]
            ).wait()

        def dst(l):
            u = ubase + l
            h = u // bph
            b0 = (u % bph) * 128
            return p_hbm.at[h, :, pl.ds(b0, 128)]

        def write_start(b, l):
            pltpu.make_async_copy(t_ring.at[b], dst(l), wsem.at[b]).start()

        def write_wait(b, l):
            pltpu.make_async_copy(t_ring.at[b], dst(l), wsem.at[b]).wait()

        def transpose_unit(b):
            @plsc.parallel_loop(0, 128, unroll=4)
            def _(i):
                col = jnp.full((16,), i, dtype=jnp.int32)
                for j in range(4):
                    vals = g_ring[b, i, pl.ds(16 * j, 16)]
                    plsc.store_scatter(t_pad, [rows8[j], col], vals)

            @plsc.parallel_loop(0, 64, unroll=4)
            def _(d):
                for j in range(8):
                    t_ring[b, d, pl.ds(16 * j, 16)] = t_pad[d, pl.ds(16 * j, 16)]

        for b in range(K2):
            gather_start(b, b)

        @pl.loop(0, upw - K2, step=K2)
        def _(l0):
            for b in range(K2):
                l = l0 + b
                gather_wait(b, l)

                @pl.when(l0 + b >= K2)
                def _():
                    write_wait(b, l)

                transpose_unit(b)
                write_start(b, l)
                gather_start(b, l + K2)

        for b in range(K2):
            l = upw - K2 + b
            gather_wait(b, l)
            write_wait(b, l)
            transpose_unit(b)
            write_start(b, l)
            write_wait(b, l)

    return p2(t2r, idx_lin)


def kernel(indices, table):
    B, H = indices.shape
    V, D = table.shape
    tt = table.T
    t2 = _phase1(tt)
    t2r = t2.reshape(V, D)
    idx_lin = indices.T.reshape(B * H // 128, 128).astype(jnp.int32)
    p = _phase2(t2r, idx_lin, H, B)
    return jnp.transpose(p, (2, 0, 1))


# speedup vs baseline: 5.7658x; 1.0813x over previous
import jax
import jax.numpy as jnp
from jax import lax
from jax.experimental import pallas as pl
from jax.experimental.pallas import tpu as pltpu
from jax.experimental.pallas import tpu_sc as plsc

NC, NS = 2, 16
NW = NC * NS
K1 = 4
K2 = 4
T2 = 2
P1 = 65
P2 = 129

_MESH = dict(core_axis_name="c", subcore_axis_name="s")


def _worker_id():
    return lax.axis_index("s") * NC + lax.axis_index("c")


def _phase1(tt):
    D, V = tt.shape
    nblk = V // 128
    tail = V % 128
    blk_w = nblk // NW
    extra = nblk % NW

    @pl.kernel(
        out_type=jax.ShapeDtypeStruct((V, 128), jnp.float32),
        mesh=plsc.VectorSubcoreMesh(**_MESH),
        scratch_types=[
            pltpu.VMEM((K1, 64, 128), jnp.float32),
            pltpu.VMEM((128 * P1,), jnp.float32),
            pltpu.VMEM((K1, 128, 128), jnp.float32),
            pltpu.SemaphoreType.DMA((K1,)),
            pltpu.SemaphoreType.DMA((K1,)),
        ],
        compiler_params=pltpu.CompilerParams(
            use_tc_tiling_on_sc=True, needs_layout_passes=False),
    )
    def p1(tt_hbm, t2_hbm, in_ring, o_pad, out_ring, rsem, wsem):
        wid = _worker_id()
        base = wid * blk_w
        iota = lax.iota(jnp.int32, 16)
        base_j = [(16 * j + iota) * P1 for j in range(8)]

        def read_start(b, blk):
            pltpu.make_async_copy(
                tt_hbm.at[:, pl.ds(pl.multiple_of(blk * 128, 128), 128)],
                in_ring.at[b], rsem.at[b]
            ).start()

        def read_wait(b, blk):
            pltpu.make_async_copy(
                tt_hbm.at[:, pl.ds(pl.multiple_of(blk * 128, 128), 128)],
                in_ring.at[b], rsem.at[b]
            ).wait()

        def write_start(b, blk):
            pltpu.make_async_copy(
                out_ring.at[b], t2_hbm.at[pl.ds(blk * 128, 128)], wsem.at[b]
            ).start()

        def write_wait(b, blk):
            pltpu.make_async_copy(
                out_ring.at[b], t2_hbm.at[pl.ds(blk * 128, 128)], wsem.at[b]
            ).wait()

        def transpose_block(b, n_vloc):
            nj = n_vloc // 16

            @plsc.parallel_loop(0, 64, unroll=4)
            def _(d):
                for j in range(nj):
                    vals = in_ring[b, d, pl.ds(16 * j, 16)]
                    plsc.store_scatter(o_pad, [base_j[j] + d], vals)

            @plsc.parallel_loop(0, n_vloc, unroll=4)
            def _(v):
                for j in range(4):
                    out_ring[b, v, pl.ds(16 * j, 16)] = o_pad[
                        pl.ds(v * P1 + 16 * j, 16)]

        for b in range(K1):
            read_start(b, base + b)

        @pl.loop(0, blk_w - K1, step=K1)
        def _(j0):
            for b in range(K1):
                blk = base + j0 + b
                read_wait(b, blk)

                @pl.when(j0 + b >= K1)
                def _():
                    write_wait(b, blk)

                transpose_block(b, 128)
                write_start(b, blk)
                read_start(b, blk + K1)

        for b in range(K1):
            blk = base + blk_w - K1 + b
            read_wait(b, blk)
            write_wait(b, blk)
            transpose_block(b, 128)
            write_start(b, blk)
            write_wait(b, blk)

        @pl.when(wid < extra)
        def _():
            blk = NW * blk_w + wid
            read_start(0, blk)
            read_wait(0, blk)
            transpose_block(0, 128)
            write_start(0, blk)
            write_wait(0, blk)

        if tail:
            @pl.when(wid == extra)
            def _():
                last = pltpu.make_async_copy(
                    tt_hbm.at[:, pl.ds(pl.multiple_of(V - tail, 128), 128)],
                    in_ring.at[1], rsem.at[1]
                )
                last.start()
                last.wait()
                transpose_block(1, tail)
                wlast = pltpu.make_async_copy(
                    out_ring.at[1, pl.ds(0, tail)],
                    t2_hbm.at[pl.ds(V - tail, tail)], wsem.at[1])
                wlast.start()
                wlast.wait()

    return p1(tt)


def _phase2(t2, idx_flat, H, B, D):
    V = t2.shape[0]
    nunit = idx_flat.shape[0] // 128
    upw = nunit // NW
    bph = B // 128

    @pl.kernel(
        out_type=jax.ShapeDtypeStruct((H, D, B), jnp.float32),
        mesh=plsc.VectorSubcoreMesh(**_MESH),
        scratch_types=[
            pltpu.VMEM((upw * 128,), jnp.int32),
            pltpu.VMEM((K2, 128, 128), jnp.float32),
            pltpu.VMEM((64 * P2,), jnp.float32),
            pltpu.VMEM((T2, 64, 128), jnp.float32),
            pltpu.SemaphoreType.DMA((K2,)),
            pltpu.SemaphoreType.DMA((T2,)),
            pltpu.SemaphoreType.DMA,
        ],
        compiler_params=pltpu.CompilerParams(
            use_tc_tiling_on_sc=True, needs_layout_passes=False),
    )
    def p2(t2_hbm, idx_hbm, p_hbm, idx_v, g_ring, t_pad, t_ring, gsem, wsem, isem):
        wid = _worker_id()
        ubase = wid * upw
        pltpu.async_copy(
            idx_hbm.at[pl.ds(ubase * 128, upw * 128)], idx_v, isem).wait()
        iota = lax.iota(jnp.int32, 16)
        base_j = [(16 * j + iota) * P2 for j in range(4)]

        def gather_start(b, l):
            pltpu.make_async_copy(
                t2_hbm.at[idx_v.at[pl.ds(l * 128, 128)]], g_ring.at[b],
                gsem.at[b]
            ).start()

        def gather_wait(b, l):
            pltpu.make_async_copy(
                t2_hbm.at[idx_v.at[pl.ds(l * 128, 128)]], g_ring.at[b],
                gsem.at[b]
            ).wait()

        def dst(l):
            u = ubase + l
            h = u // bph
            b0 = pl.multiple_of((u % bph) * 128, 128)
            return p_hbm.at[h, :, pl.ds(b0, 128)]

        def write_start(b, l):
            pltpu.make_async_copy(t_ring.at[b], dst(l), wsem.at[b]).start()

        def write_wait(b, l):
            pltpu.make_async_copy(t_ring.at[b], dst(l), wsem.at[b]).wait()

        def transpose_unit(b, w):
            @plsc.parallel_loop(0, 128, unroll=4)
            def _(i):
                col = jnp.full((16,), i, dtype=jnp.int32)
                for j in range(4):
                    vals = g_ring[b, i, pl.ds(16 * j, 16)]
                    plsc.store_scatter(t_pad, [base_j[j] + col], vals)

            @plsc.parallel_loop(0, 64, unroll=4)
            def _(d):
                for j in range(8):
                    t_ring[w, d, pl.ds(16 * j, 16)] = t_pad[
                        pl.ds(d * P2 + 16 * j, 16)]

        for b in range(K2):
            gather_start(b, b)

        @pl.loop(0, upw - K2, step=K2)
        def _(l0):
            for b in range(K2):
                l = l0 + b
                w = b % T2
                gather_wait(b, l)

                @pl.when(l0 + b >= T2)
                def _():
                    write_wait(w, l)

                transpose_unit(b, w)
                write_start(w, l)
                gather_start(b, l + K2)

        for b in range(K2):
            l = upw - K2 + b
            w = b % T2
            gather_wait(b, l)
            write_wait(w, l)
            transpose_unit(b, w)
            write_start(w, l)

        for w in range(T2):
            write_wait(w, upw - T2 + w)

    return p2(t2, idx_flat)


def kernel(indices, table):
    B, H = indices.shape
    V, D = table.shape
    tt = table.T
    t2 = _phase1(tt)
    idx_flat = indices.T.reshape(B * H).astype(jnp.int32)
    p = _phase2(t2, idx_flat, H, B, D)
    return jnp.transpose(p, (2, 0, 1))
